# baseline (speedup 1.0000x reference)
import functools

import jax
import jax.numpy as jnp
from jax import lax
from jax.experimental import pallas as pl
from jax.experimental.pallas import tpu as pltpu
from jax.experimental.pallas import tpu_sc as plsc

F = 26
V = 100000
D = 32
B = 16384

NC = 2
NS = 16
NW = NC * NS
ROWS = B * F
ROWS_W = ROWS // NW
CHUNK = 1024
NCHUNK = ROWS_W // CHUNK
SUB = 256
NSUB = CHUNK // SUB
VPC = CHUNK // 16
OPS = SUB // 4


def _sc_gather(idx2d, tab128):
    mesh = plsc.VectorSubcoreMesh(core_axis_name="c", subcore_axis_name="s")

    @functools.partial(
        pl.kernel,
        mesh=mesh,
        out_type=jax.ShapeDtypeStruct((ROWS // 4, 128), jnp.float32),
        compiler_params=pltpu.CompilerParams(
            use_tc_tiling_on_sc=True, needs_layout_passes=False),
        scratch_types=[
            pltpu.VMEM((2, CHUNK // 128, 128), jnp.int32),
            pltpu.VMEM((2, CHUNK // 128, 128), jnp.int32),
            pltpu.VMEM((2, CHUNK // 128, 128), jnp.int32),
            pltpu.VMEM((2, SUB, 128), jnp.float32),
            pltpu.VMEM((2, OPS, 128), jnp.float32),
            pltpu.SemaphoreType.DMA,
            pltpu.SemaphoreType.DMA,
            pltpu.SemaphoreType.DMA,
            pltpu.SemaphoreType.DMA,
            pltpu.SemaphoreType.DMA,
            pltpu.SemaphoreType.DMA,
        ],
    )
    def k(idx_hbm, tab_hbm, out_hbm, idx_v, g_v, s_v, prow_v, orow_v,
          idx_s0, idx_s1, gat_s0, gat_s1, out_s0, out_s1):
        wid = lax.axis_index("s") * NC + lax.axis_index("c")
        irow0 = wid * (ROWS_W // 128)
        orow0 = wid * (ROWS_W // 4)
        lane = lax.broadcasted_iota(jnp.int32, (16,), 0)

        idx_sems = (idx_s0, idx_s1)
        gat_sems = (gat_s0, gat_s1)
        out_sems = (out_s0, out_s1)

        def idx_start(cc, bi):
            pltpu.async_copy(
                idx_hbm.at[pl.ds(irow0 + cc * 8, 8)], idx_v.at[bi],
                idx_sems[bi])

        def idx_wait(bi):
            pltpu.make_async_copy(
                idx_hbm.at[pl.ds(irow0, 8)], idx_v.at[bi],
                idx_sems[bi]).wait()

        def out_wait(b2):
            pltpu.make_async_copy(
                orow_v.at[b2], out_hbm.at[pl.ds(orow0, OPS)],
                out_sems[b2]).wait()

        def splat(vec, l):
            return lax.gather(
                vec, jnp.full((16, 1), l, jnp.int32),
                lax.GatherDimensionNumbers(
                    offset_dims=(), collapsed_slice_dims=(0,),
                    start_index_map=(0,)),
                (1,), mode=lax.GatherScatterMode.PROMISE_IN_BOUNDS)

        def process_chunk(cc, bi, skip_first_out_waits):
            idx_wait(bi)

            def body(v, carry):
                j = v // 8
                col = (v % 8) * 16
                pos = cc * CHUNK + v * 16 + lane
                gidx = idx_v[bi, j, pl.ds(col, 16)] + (pos % F) * V
                g_v[bi, j, pl.ds(col, 16)] = lax.shift_right_logical(gidx, 2)
                s_v[bi, j, pl.ds(col, 16)] = (gidx & 3) * D
                return carry

            lax.fori_loop(0, VPC, body, 0)

            for u in range(NSUB):
                b2 = u & 1
                if not (skip_first_out_waits and u < 2):
                    out_wait(b2)
                gats = [
                    pltpu.async_copy(
                        tab_hbm.at[g_v.at[bi, u * 2 + j]],
                        prow_v.at[b2, pl.ds(j * 128, 128)], gat_sems[b2])
                    for j in range(2)
                ]
                for g in gats:
                    g.wait()

                def ebody(grp, carry):
                    p0 = u * SUB + grp * 16
                    sv = s_v[bi, p0 // 128, pl.ds(p0 % 128, 16)]
                    for l in range(16):
                        i = grp * 16 + l
                        soff = splat(sv, l)
                        rowv = jnp.full((16,), i, jnp.int32)
                        c0 = soff + lane
                        for kk in range(2):
                            vals = plsc.load_gather(
                                prow_v.at[b2], [rowv, c0 + kk * 16])
                            orow_v[b2, i // 4,
                                   pl.ds((i % 4) * D + kk * 16, 16)] = vals
                    return carry

                lax.fori_loop(0, SUB // 16, ebody, 0)

                pltpu.async_copy(
                    orow_v.at[b2],
                    out_hbm.at[pl.ds(orow0 + (cc * NSUB + u) * OPS, OPS)],
                    out_sems[b2])

        idx_start(0, 0)
        idx_start(1, 1)
        process_chunk(0, 0, skip_first_out_waits=True)

        def pair_body(t, carry):
            for off in range(2):
                cc = 2 * t + 1 + off
                bi = 1 - off
                nxt = cc + 1

                @pl.when(nxt < NCHUNK)
                def _():
                    idx_start(nxt, 1 - bi)

                process_chunk(cc, bi, skip_first_out_waits=False)
            return carry

        lax.fori_loop(0, (NCHUNK - 1) // 2, pair_body, 0)
        out_wait(0)
        out_wait(1)

    return k(idx2d, tab128)


def kernel(x_cat, tables):
    idx2d = x_cat.reshape(ROWS // 128, 128)
    tab128 = tables.reshape(F * V // 4, 128)
    out = _sc_gather(idx2d, tab128)
    return out.reshape(B, F * D)
